# baseline (speedup 1.0000x reference)
import functools

import jax
import jax.numpy as jnp
from jax import lax
from jax.experimental import pallas as pl
from jax.experimental.pallas import tpu as pltpu
from jax.experimental.pallas import tpu_sc as plsc


def _lse_block(x_ref, o_ref):
    x = x_ref[...]
    m = jnp.max(x, axis=1)
    s = jnp.sum(jnp.exp(x - m[:, None]), axis=1)
    o_ref[0, 0, :] = m + jnp.log(s)


def _row_lse(table):
    v, c = table.shape
    rows = 256
    grid = v // rows
    out = pl.pallas_call(
        _lse_block,
        grid=(grid,),
        in_specs=[pl.BlockSpec((rows, c), lambda i: (i, 0))],
        out_specs=pl.BlockSpec((1, 1, rows), lambda i: (i, 0, 0)),
        out_shape=jax.ShapeDtypeStruct((grid, 1, rows), jnp.float32),
    )(table)
    return out.reshape(-1)


def _make_sc_gather(n, vocab, c, nc, ns, lanes):
    nw = nc * ns
    npw = n // nw
    cha = 16
    chb = 8
    per = cha + chb
    nper = npw // per
    nrem = npw - nper * per
    gch = 128
    ngch = npw // gch
    mesh = plsc.VectorSubcoreMesh(core_axis_name="c", subcore_axis_name="s")

    @functools.partial(
        pl.kernel,
        mesh=mesh,
        out_type=[
            jax.ShapeDtypeStruct((n, c), jnp.float32),
            jax.ShapeDtypeStruct((nw, lanes), jnp.float32),
        ],
        scratch_types=[
            pltpu.VMEM((npw,), jnp.int32),
            pltpu.VMEM((npw + lanes,), jnp.int32),
            pltpu.VMEM((npw + lanes,), jnp.float32),
            pltpu.VMEM((cha, c), jnp.float32),
            pltpu.VMEM((chb, c), jnp.float32),
            pltpu.VMEM((lanes,), jnp.float32),
            pltpu.SemaphoreType.DMA,
            pltpu.SemaphoreType.DMA,
            pltpu.SemaphoreType.DMA,
            pltpu.SemaphoreType.DMA,
            pltpu.SemaphoreType.DMA,
        ],
        compiler_params=pltpu.CompilerParams(needs_layout_passes=False),
    )
    def sc_kernel(idx_hbm, tgt_hbm, table_hbm, lse_hbm, out_hbm, part_hbm,
                  idx_v, tgt_v, lsev_v, buf_a, buf_b, acc_v,
                  semga, semgb, semsa, semsb, semx):
        cid = lax.axis_index("c")
        sid = lax.axis_index("s")
        wid = sid * nc + cid
        base = wid * npw
        pltpu.sync_copy(idx_hbm.at[pl.ds(base, npw)], idx_v)
        pltpu.sync_copy(tgt_hbm.at[pl.ds(base, npw)], tgt_v.at[pl.ds(0, npw)])
        lane = lax.iota(jnp.int32, lanes)
        zero16i = jnp.zeros((lanes,), jnp.int32)
        tgt_v[pl.ds(npw, lanes)] = zero16i

        for j in range(ngch):
            pltpu.async_copy(
                lse_hbm.at[idx_v.at[pl.ds(j * gch, gch)]],
                lsev_v.at[pl.ds(j * gch, gch)], semx)
        for j in range(ngch):
            pltpu.make_async_copy(
                lse_hbm.at[idx_v.at[pl.ds(j * gch, gch)]],
                lsev_v.at[pl.ds(j * gch, gch)], semx).wait()

        def start_gather(buf, ch, semg, off):
            off = pl.multiple_of(off, 8)
            pltpu.async_copy(
                table_hbm.at[idx_v.at[pl.ds(off, ch)]], buf, semg)

        def wait_gather(buf, ch, semg, off):
            off = pl.multiple_of(off, 8)
            pltpu.make_async_copy(
                table_hbm.at[idx_v.at[pl.ds(off, ch)]], buf, semg).wait()

        def start_store(buf, ch, sems, off):
            off = pl.multiple_of(off, 8)
            pltpu.async_copy(buf, out_hbm.at[pl.ds(base + off, ch)], sems)

        def wait_store(buf, ch, sems, off):
            off = pl.multiple_of(off, 8)
            pltpu.make_async_copy(buf, out_hbm.at[pl.ds(base + off, ch)],
                                  sems).wait()

        def loss_a(off, acc):
            off = pl.multiple_of(off, 8)
            tgt16 = tgt_v[pl.ds(off, lanes)]
            lse16 = lsev_v[pl.ds(off, lanes)]
            tval16 = plsc.load_gather(buf_a, [lane, tgt16])
            return acc + (lse16 - tval16)

        def loss_b(off, acc):
            off = pl.multiple_of(off, 8)
            tgt16 = tgt_v[pl.ds(off, lanes)]
            lse16 = lsev_v[pl.ds(off, lanes)]
            row16 = jnp.bitwise_and(lane, chb - 1)
            tval16 = plsc.load_gather(buf_b, [row16, tgt16])
            return acc + jnp.where(lane < chb, lse16 - tval16, 0.0)

        start_gather(buf_a, cha, semga, 0)
        start_gather(buf_b, chb, semgb, cha)

        def body(k, acc):
            off = k * per
            wait_gather(buf_a, cha, semga, off)
            start_store(buf_a, cha, semsa, off)
            acc = loss_a(off, acc)
            wait_gather(buf_b, chb, semgb, off + cha)
            start_store(buf_b, chb, semsb, off + cha)
            acc = loss_b(off + cha, acc)

            @pl.when(k < nper - 1)
            def _():
                wait_store(buf_a, cha, semsa, off)
                start_gather(buf_a, cha, semga, off + per)
                wait_store(buf_b, chb, semsb, off + cha)
                start_gather(buf_b, chb, semgb, off + per + cha)

            @pl.when(k == nper - 1)
            def _():
                wait_store(buf_b, chb, semsb, off + cha)
                start_gather(buf_b, chb, semgb, off + per)
            return acc

        acc = lax.fori_loop(0, nper, body, jnp.zeros((lanes,), jnp.float32))
        rem = nper * per
        wait_gather(buf_b, chb, semgb, rem)
        start_store(buf_b, chb, semsb, rem)
        acc = loss_b(rem, acc)
        wait_store(buf_a, cha, semsa, (nper - 1) * per)
        wait_store(buf_b, chb, semsb, rem)
        acc_v[...] = acc
        pltpu.sync_copy(acc_v, part_hbm.at[wid])

    return sc_kernel


def kernel(index, targets, table):
    b, t = index.shape
    vocab, c = table.shape
    n = b * t
    idx = index.reshape(-1).astype(jnp.int32)
    tgt = targets.reshape(-1).astype(jnp.int32)
    lse = _row_lse(table)
    info = plsc.get_sparse_core_info()
    nc, ns, lanes = info.num_cores, info.num_subcores, info.num_lanes
    sc = _make_sc_gather(n, vocab, c, nc, ns, lanes)
    logits, partials = sc(idx, tgt, table, lse)
    loss = jnp.sum(partials) / n
    return (logits, loss)


# speedup vs baseline: 1.0515x; 1.0515x over previous
import functools

import jax
import jax.numpy as jnp
from jax import lax
from jax.experimental import pallas as pl
from jax.experimental.pallas import tpu as pltpu
from jax.experimental.pallas import tpu_sc as plsc


def _lse_block(x_ref, o_ref):
    x = x_ref[...]
    m = jnp.max(x, axis=1)
    s = jnp.sum(jnp.exp(x - m[:, None]), axis=1)
    o_ref[0, 0, :] = m + jnp.log(s)


def _row_lse(table):
    v, c = table.shape
    rows = 256
    grid = v // rows
    out = pl.pallas_call(
        _lse_block,
        grid=(grid,),
        in_specs=[pl.BlockSpec((rows, c), lambda i: (i, 0))],
        out_specs=pl.BlockSpec((1, 1, rows), lambda i: (i, 0, 0)),
        out_shape=jax.ShapeDtypeStruct((grid, 1, rows), jnp.float32),
    )(table)
    return out.reshape(-1)


def _make_sc_gather(n, vocab, c, nc, ns, lanes):
    nw = nc * ns
    npw = n // nw
    ch = 8
    nch = npw // ch
    nbuf = 3
    gch = 128
    ngch = npw // gch
    ngrp = nch // nbuf
    nrem = nch - ngrp * nbuf
    mesh = plsc.VectorSubcoreMesh(core_axis_name="c", subcore_axis_name="s")

    @functools.partial(
        pl.kernel,
        mesh=mesh,
        out_type=[
            jax.ShapeDtypeStruct((n, c), jnp.float32),
            jax.ShapeDtypeStruct((nw, lanes), jnp.float32),
            jax.ShapeDtypeStruct((nc, vocab), jnp.float32),
        ],
        scratch_types=[
            pltpu.VMEM((npw,), jnp.int32),
            pltpu.VMEM((ngch, gch), jnp.int32),
            pltpu.VMEM((npw + lanes,), jnp.int32),
            pltpu.VMEM((gch,), jnp.float32),
            pltpu.VMEM((vocab,), jnp.float32),
            [pltpu.VMEM((ch, c), jnp.float32) for _ in range(nbuf)],
            pltpu.VMEM((lanes,), jnp.float32),
            pltpu.VMEM_SHARED((vocab,), jnp.float32),
            [pltpu.SemaphoreType.DMA for _ in range(nbuf)],
            [pltpu.SemaphoreType.DMA for _ in range(nbuf)],
            pltpu.SemaphoreType.DMA,
        ],
        compiler_params=pltpu.CompilerParams(needs_layout_passes=False),
    )
    def sc_kernel(idx_hbm, tgt_hbm, table_hbm, out_hbm, part_hbm, cnt_hbm,
                  idx_v, idx2_v, tgt_v, ones_v, stage_v, bufs, acc_v,
                  hist_s, semg, sems, semx):
        cid = lax.axis_index("c")
        sid = lax.axis_index("s")
        wid = sid * nc + cid
        base = wid * npw
        pltpu.sync_copy(idx_hbm.at[pl.ds(base, npw)], idx_v)
        pltpu.sync_copy(tgt_hbm.at[pl.ds(base, npw)], tgt_v.at[pl.ds(0, npw)])
        lane = lax.iota(jnp.int32, lanes)
        tgt_v[pl.ds(npw, lanes)] = jnp.zeros((lanes,), jnp.int32)

        one16 = jnp.full((lanes,), 1.0, jnp.float32)
        zero16 = jnp.zeros((lanes,), jnp.float32)
        for j in range(gch // lanes):
            ones_v[pl.ds(j * lanes, lanes)] = one16
        for j in range(vocab // lanes):
            stage_v[pl.ds(j * lanes, lanes)] = zero16

        @pl.when(sid == 0)
        def _():
            pltpu.sync_copy(stage_v, hist_s)

        for j in range(ngch):
            pltpu.sync_copy(idx_hbm.at[pl.ds(base + j * gch, gch)],
                            idx2_v.at[j])
        plsc.subcore_barrier()
        for j in range(ngch):
            pltpu.sync_copy(ones_v, hist_s.at[idx2_v.at[j]], add=True)
        plsc.subcore_barrier()

        @pl.when(sid == 0)
        def _():
            pltpu.sync_copy(hist_s, cnt_hbm.at[cid])

        def start_gather(b, ci):
            off = pl.multiple_of(ci * ch, 8)
            pltpu.async_copy(
                table_hbm.at[idx_v.at[pl.ds(off, ch)]], bufs[b], semg[b])

        def wait_gather(b, ci):
            off = pl.multiple_of(ci * ch, 8)
            pltpu.make_async_copy(
                table_hbm.at[idx_v.at[pl.ds(off, ch)]], bufs[b],
                semg[b]).wait()

        def start_store(b, ci):
            off = pl.multiple_of(ci * ch, 8)
            pltpu.async_copy(bufs[b], out_hbm.at[pl.ds(base + off, ch)],
                             sems[b])

        def wait_store(b, ci):
            off = pl.multiple_of(ci * ch, 8)
            pltpu.make_async_copy(bufs[b], out_hbm.at[pl.ds(base + off, ch)],
                                  sems[b]).wait()

        def chunk_tval(b, ci, acc):
            off = pl.multiple_of(ci * ch, 8)
            tgt16 = tgt_v[pl.ds(off, lanes)]
            row16 = jnp.bitwise_and(lane, ch - 1)
            tval16 = plsc.load_gather(bufs[b], [row16, tgt16])
            return acc + jnp.where(lane < ch, tval16, 0.0)

        for b in range(nbuf):
            start_gather(b, b)

        def body(k, acc):
            for b in range(nbuf):
                ci = k * nbuf + b
                wait_gather(b, ci)
                start_store(b, ci)
                acc = chunk_tval(b, ci, acc)
            for b in range(nbuf):
                ci = k * nbuf + b

                @pl.when(ci + nbuf < nch)
                def _():
                    wait_store(b, ci)
                    start_gather(b, ci + nbuf)
            return acc

        acc = lax.fori_loop(0, ngrp, body, jnp.zeros((lanes,), jnp.float32))
        for r in range(nrem):
            ci = ngrp * nbuf + r
            wait_gather(r, ci)
            start_store(r, ci)
            acc = chunk_tval(r, ci, acc)
        for b in range(nbuf):
            if b < nrem:
                wait_store(b, ngrp * nbuf + b)
            else:
                wait_store(b, (ngrp - 1) * nbuf + b)
        acc_v[...] = acc
        pltpu.sync_copy(acc_v, part_hbm.at[wid])

    return sc_kernel


def kernel(index, targets, table):
    b, t = index.shape
    vocab, c = table.shape
    n = b * t
    idx = index.reshape(-1).astype(jnp.int32)
    tgt = targets.reshape(-1).astype(jnp.int32)
    lse = _row_lse(table)
    info = plsc.get_sparse_core_info()
    nc, ns, lanes = info.num_cores, info.num_subcores, info.num_lanes
    sc = _make_sc_gather(n, vocab, c, nc, ns, lanes)
    logits, tpart, counts = sc(idx, tgt, table)
    loss = (jnp.dot(jnp.sum(counts, axis=0), lse) - jnp.sum(tpart)) / n
    return (logits, loss)


# speedup vs baseline: 1.0541x; 1.0025x over previous
import functools

import jax
import jax.numpy as jnp
from jax import lax
from jax.experimental import pallas as pl
from jax.experimental.pallas import tpu as pltpu
from jax.experimental.pallas import tpu_sc as plsc


def _lse_block(x_ref, o_ref):
    x = x_ref[...]
    m = jnp.max(x, axis=1)
    s = jnp.sum(jnp.exp(x - m[:, None]), axis=1)
    o_ref[0, 0, :] = m + jnp.log(s)


def _row_lse(table):
    v, c = table.shape
    rows = 256
    grid = v // rows
    out = pl.pallas_call(
        _lse_block,
        grid=(grid,),
        in_specs=[pl.BlockSpec((rows, c), lambda i: (i, 0))],
        out_specs=pl.BlockSpec((1, 1, rows), lambda i: (i, 0, 0)),
        out_shape=jax.ShapeDtypeStruct((grid, 1, rows), jnp.float32),
    )(table)
    return out.reshape(-1)


def _make_sc_gather(n, vocab, c, nc, ns, lanes):
    nw = nc * ns
    npw = n // nw
    ch = 8
    nch = npw // ch
    nbuf = 3
    gch = 128
    ngch = npw // gch
    ngrp = nch // nbuf
    nrem = nch - ngrp * nbuf
    mesh = plsc.VectorSubcoreMesh(core_axis_name="c", subcore_axis_name="s")

    @functools.partial(
        pl.kernel,
        mesh=mesh,
        out_type=[
            jax.ShapeDtypeStruct((n, c), jnp.float32),
            jax.ShapeDtypeStruct((nw, lanes), jnp.float32),
            jax.ShapeDtypeStruct((nc, vocab), jnp.float32),
        ],
        scratch_types=[
            pltpu.VMEM((npw,), jnp.int32),
            pltpu.VMEM((ngch, gch), jnp.int32),
            pltpu.VMEM((npw + lanes,), jnp.int32),
            pltpu.VMEM((gch,), jnp.float32),
            pltpu.VMEM((vocab,), jnp.float32),
            [pltpu.VMEM((ch, c), jnp.float32) for _ in range(nbuf)],
            pltpu.VMEM((lanes,), jnp.float32),
            pltpu.VMEM_SHARED((vocab,), jnp.float32),
            [pltpu.SemaphoreType.DMA for _ in range(nbuf)],
            [pltpu.SemaphoreType.DMA for _ in range(nbuf)],
            pltpu.SemaphoreType.DMA,
        ],
        compiler_params=pltpu.CompilerParams(needs_layout_passes=False),
    )
    def sc_kernel(idx_hbm, tgt_hbm, table_hbm, out_hbm, part_hbm, cnt_hbm,
                  idx_v, idx2_v, tgt_v, ones_v, stage_v, bufs, acc_v,
                  hist_s, semg, sems, semx):
        cid = lax.axis_index("c")
        sid = lax.axis_index("s")
        wid = sid * nc + cid
        base = wid * npw
        pltpu.sync_copy(idx_hbm.at[pl.ds(base, npw)], idx_v)
        pltpu.sync_copy(tgt_hbm.at[pl.ds(base, npw)], tgt_v.at[pl.ds(0, npw)])
        lane = lax.iota(jnp.int32, lanes)
        tgt_v[pl.ds(npw, lanes)] = jnp.zeros((lanes,), jnp.int32)

        def start_gather(b, ci):
            off = pl.multiple_of(ci * ch, 8)
            pltpu.async_copy(
                table_hbm.at[idx_v.at[pl.ds(off, ch)]], bufs[b], semg[b])

        def wait_gather(b, ci):
            off = pl.multiple_of(ci * ch, 8)
            pltpu.make_async_copy(
                table_hbm.at[idx_v.at[pl.ds(off, ch)]], bufs[b],
                semg[b]).wait()

        def start_store(b, ci):
            off = pl.multiple_of(ci * ch, 8)
            pltpu.async_copy(bufs[b], out_hbm.at[pl.ds(base + off, ch)],
                             sems[b])

        def wait_store(b, ci):
            off = pl.multiple_of(ci * ch, 8)
            pltpu.make_async_copy(bufs[b], out_hbm.at[pl.ds(base + off, ch)],
                                  sems[b]).wait()

        def chunk_tval(b, ci, acc):
            off = pl.multiple_of(ci * ch, 8)
            tgt16 = tgt_v[pl.ds(off, lanes)]
            row16 = jnp.bitwise_and(lane, ch - 1)
            tval16 = plsc.load_gather(bufs[b], [row16, tgt16])
            return acc + jnp.where(lane < ch, tval16, 0.0)

        for b in range(nbuf):
            start_gather(b, b)

        one16 = jnp.full((lanes,), 1.0, jnp.float32)
        zero16 = jnp.zeros((lanes,), jnp.float32)
        for j in range(gch // lanes):
            ones_v[pl.ds(j * lanes, lanes)] = one16
        for j in range(vocab // lanes):
            stage_v[pl.ds(j * lanes, lanes)] = zero16

        @pl.when(sid == 0)
        def _():
            pltpu.sync_copy(stage_v, hist_s)

        for j in range(ngch):
            pltpu.sync_copy(idx_hbm.at[pl.ds(base + j * gch, gch)],
                            idx2_v.at[j])
        plsc.subcore_barrier()
        for j in range(ngch):
            pltpu.sync_copy(ones_v, hist_s.at[idx2_v.at[j]], add=True)

        def body(k, acc):
            for b in range(nbuf):
                ci = k * nbuf + b
                wait_gather(b, ci)
                start_store(b, ci)
                acc = chunk_tval(b, ci, acc)
            for b in range(nbuf):
                ci = k * nbuf + b

                @pl.when(ci + nbuf < nch)
                def _():
                    wait_store(b, ci)
                    start_gather(b, ci + nbuf)
            return acc

        acc = lax.fori_loop(0, ngrp, body, jnp.zeros((lanes,), jnp.float32))
        for r in range(nrem):
            ci = ngrp * nbuf + r
            wait_gather(r, ci)
            start_store(r, ci)
            acc = chunk_tval(r, ci, acc)
        for b in range(nbuf):
            if b < nrem:
                wait_store(b, ngrp * nbuf + b)
            else:
                wait_store(b, (ngrp - 1) * nbuf + b)
        acc_v[...] = acc
        pltpu.sync_copy(acc_v, part_hbm.at[wid])
        plsc.subcore_barrier()

        @pl.when(sid == 0)
        def _():
            pltpu.sync_copy(hist_s, cnt_hbm.at[cid])

    return sc_kernel


def kernel(index, targets, table):
    b, t = index.shape
    vocab, c = table.shape
    n = b * t
    idx = index.reshape(-1).astype(jnp.int32)
    tgt = targets.reshape(-1).astype(jnp.int32)
    info = plsc.get_sparse_core_info()
    nc, ns, lanes = info.num_cores, info.num_subcores, info.num_lanes
    sc = _make_sc_gather(n, vocab, c, nc, ns, lanes)
    logits, tpart, counts = sc(idx, tgt, table)
    lse = _row_lse(table)
    loss = (jnp.dot(jnp.sum(counts, axis=0), lse) - jnp.sum(tpart)) / n
    return (logits, loss)
